# baseline (speedup 1.0000x reference)
import functools

import jax
import jax.numpy as jnp
import numpy as np
from jax import lax
from jax.experimental import pallas as pl
from jax.experimental.pallas import tpu as pltpu
from jax.experimental.pallas import tpu_sc as plsc

ROWS = 128
COLS = 32768
NCHUNK = COLS // 16
R1 = 26213
R2 = 26214
FRAC = 0.599609375

NTILES = 32
ROWS_PER_TILE = ROWS // NTILES

_SIGNMASK = 0x7FFFFFFF


def _i32const(v):
    return jnp.int32(np.uint32(v & 0xFFFFFFFF).astype(np.int32))


def _sc_quantile(x):
    mesh = plsc.VectorSubcoreMesh(core_axis_name="c", subcore_axis_name="s")

    @functools.partial(
        pl.kernel,
        mesh=mesh,
        compiler_params=pltpu.CompilerParams(needs_layout_passes=False),
        out_type=jax.ShapeDtypeStruct((NTILES, 16), jnp.float32),
        scratch_types=[
            pltpu.VMEM((COLS,), jnp.float32),
            pltpu.VMEM((COLS,), jnp.int32),
            pltpu.VMEM((16 * 273,), jnp.int32),
            pltpu.VMEM((16,), jnp.float32),
        ],
    )
    def sc_kernel(x_hbm, out_hbm, row_v, key_v, hist_v, mv_v):
        wid = lax.axis_index("c") * 16 + lax.axis_index("s")
        lanes = lax.iota(jnp.int32, 16)
        laneoff = lanes * 273
        ones = jnp.full((16,), 1, jnp.int32)
        zeros_i = jnp.zeros((16,), jnp.int32)
        mvec = jnp.zeros((16,), jnp.float32)

        def zero_hist():
            @plsc.parallel_loop(0, 273, unroll=8)
            def _(i):
                hist_v[pl.ds(i * 16, 16)] = zeros_i

        def scan_hist(r):
            z = jnp.int32(0)

            @plsc.parallel_loop(0, 16, unroll=2, carry=(z, z, z, z))
            def scan_body(cb, carry):
                run, D, cbef, tD = carry
                t = hist_v[pl.ds(cb * 16, 16)]
                for l in range(1, 16):
                    t = t + hist_v[pl.ds(l * 273 + cb * 16, 16)]
                c = plsc.cumsum(t) + run
                le = c <= r
                D = D + jnp.sum(jnp.where(le, ones, zeros_i))
                cbef = cbef + jnp.sum(jnp.where(le, t, zeros_i))
                cross = jnp.logical_and(c > r, (c - t) <= r)
                tD = tD + jnp.sum(jnp.where(cross, t, zeros_i))
                run = run + jnp.sum(t)
                return run, D, cbef, tD

            _, D, cbef, tD = scan_body
            return D, cbef, tD

        for j in range(ROWS_PER_TILE):
            row = wid * ROWS_PER_TILE + j
            pltpu.sync_copy(x_hbm.at[row], row_v)

            zero_hist()

            @plsc.parallel_loop(0, NCHUNK, unroll=8)
            def _(i):
                off = i * 16
                v = row_v[pl.ds(off, 16)]
                b = lax.bitcast_convert_type(v, jnp.int32)
                kk = b ^ ((b >> 31) & _SIGNMASK)
                key_v[pl.ds(off, 16)] = kk
                dig = ((kk >> 24) & 0xFF) ^ 0x80
                plsc.addupdate_scatter(hist_v, [laneoff + dig], ones)

            r = jnp.int32(R1)
            D, cbef, tD = scan_hist(r)
            acc = (D ^ 0x80) << 24
            r = r - cbef
            less = cbef

            for level in (1, 2, 3):
                shift = 24 - 8 * level
                mbits = _i32const(0xFFFFFFFF << (shift + 8))
                zero_hist()

                @plsc.parallel_loop(0, NCHUNK, unroll=8)
                def _(i, shift=shift, mbits=mbits, acc=acc):
                    kk = key_v[pl.ds(i * 16, 16)]
                    ing = (kk & mbits) == acc
                    dig = (kk >> shift) & 0xFF
                    plsc.addupdate_scatter(
                        hist_v, [laneoff + dig], ones, mask=ing)
                D, cbef, tD = scan_hist(r)
                acc = acc | (D << shift)
                r = r - cbef
                less = less + cbef

            key_a = acc
            cnt_le = less + tD

            big = jnp.full((16,), 0x7FFFFFFF, jnp.int32)

            @plsc.parallel_loop(0, NCHUNK, unroll=8, carry=big)
            def nm_body(i, acc_v, key_a=key_a):
                kk = key_v[pl.ds(i * 16, 16)]
                return jnp.minimum(acc_v, jnp.where(kk > key_a, kk, big))

            accv = nm_body
            key_b = jnp.min(accv)
            key_b = jnp.where(cnt_le >= jnp.int32(R2 + 1), key_a, key_b)

            va = lax.bitcast_convert_type(
                key_a ^ ((key_a >> 31) & _SIGNMASK), jnp.float32)
            vb = lax.bitcast_convert_type(
                key_b ^ ((key_b >> 31) & _SIGNMASK), jnp.float32)
            m = va + (vb - va) * jnp.float32(FRAC)
            mvec = jnp.where(lanes == j, m, mvec)

        mv_v[...] = mvec
        pltpu.sync_copy(mv_v, out_hbm.at[wid])

    return sc_kernel(x)


def _tc_body(x_ref, m_ref, o_ref):
    o_ref[...] = jnp.maximum(x_ref[...] - m_ref[...], 0.0) + 1.0


def _tc_mask(x, m):
    return pl.pallas_call(
        _tc_body,
        grid=(16, 8),
        in_specs=[
            pl.BlockSpec((8, COLS // 8), lambda i, j: (i, j)),
            pl.BlockSpec((8, 1), lambda i, j: (i, 0)),
        ],
        out_specs=pl.BlockSpec((8, COLS // 8), lambda i, j: (i, j)),
        out_shape=jax.ShapeDtypeStruct((ROWS, COLS), jnp.float32),
    )(x, m)


@jax.jit
def kernel(x):
    mq = _sc_quantile(x)
    m = mq[:, :ROWS_PER_TILE].reshape(ROWS, 1)
    return _tc_mask(x, m)


# speedup vs baseline: 1.6123x; 1.6123x over previous
import functools

import jax
import jax.numpy as jnp
import numpy as np
from jax import lax
from jax.experimental import pallas as pl
from jax.experimental.pallas import tpu as pltpu
from jax.experimental.pallas import tpu_sc as plsc

ROWS = 128
COLS = 32768
NCHUNK = COLS // 16
R1 = 26213
R2 = 26214
FRAC = 0.599609375

NTILES = 32
ROWS_PER_TILE = ROWS // NTILES

_SIGNMASK = 0x7FFFFFFF


def _i32const(v):
    return jnp.int32(np.uint32(v & 0xFFFFFFFF).astype(np.int32))


def _sc_quantile_mask(x):
    mesh = plsc.VectorSubcoreMesh(core_axis_name="c", subcore_axis_name="s")

    @functools.partial(
        pl.kernel,
        mesh=mesh,
        compiler_params=pltpu.CompilerParams(needs_layout_passes=False),
        out_type=jax.ShapeDtypeStruct((ROWS, COLS), jnp.float32),
        scratch_types=[
            pltpu.VMEM((COLS,), jnp.float32),
            pltpu.VMEM((COLS,), jnp.int32),
            pltpu.VMEM((16 * 273,), jnp.int32),
        ],
    )
    def sc_kernel(x_hbm, out_hbm, row_v, key_v, hist_v):
        wid = lax.axis_index("c") * 16 + lax.axis_index("s")
        lanes = lax.iota(jnp.int32, 16)
        laneoff = lanes * 273
        ones = jnp.full((16,), 1, jnp.int32)
        zeros_i = jnp.zeros((16,), jnp.int32)

        def zero_hist():
            @plsc.parallel_loop(0, 273, unroll=8)
            def _(i):
                hist_v[pl.ds(i * 16, 16)] = zeros_i

        def scan_hist(r):
            z = jnp.int32(0)

            @plsc.parallel_loop(0, 16, unroll=2, carry=(z, z, z, z))
            def scan_body(cb, carry):
                run, D, cbef, tD = carry
                t = hist_v[pl.ds(cb * 16, 16)]
                for l in range(1, 16):
                    t = t + hist_v[pl.ds(l * 273 + cb * 16, 16)]
                c = plsc.cumsum(t) + run
                le = c <= r
                D = D + jnp.sum(jnp.where(le, ones, zeros_i))
                cbef = cbef + jnp.sum(jnp.where(le, t, zeros_i))
                cross = jnp.logical_and(c > r, (c - t) <= r)
                tD = tD + jnp.sum(jnp.where(cross, t, zeros_i))
                run = run + jnp.sum(t)
                return run, D, cbef, tD

            _, D, cbef, tD = scan_body
            return D, cbef, tD

        for j in range(ROWS_PER_TILE):
            row = wid * ROWS_PER_TILE + j
            pltpu.sync_copy(x_hbm.at[row], row_v)

            zero_hist()

            @plsc.parallel_loop(0, NCHUNK, unroll=8)
            def _(i):
                off = i * 16
                v = row_v[pl.ds(off, 16)]
                b = lax.bitcast_convert_type(v, jnp.int32)
                kk = b ^ ((b >> 31) & _SIGNMASK)
                key_v[pl.ds(off, 16)] = kk
                dig = ((kk >> 24) & 0xFF) ^ 0x80
                plsc.addupdate_scatter(hist_v, [laneoff + dig], ones)

            r = jnp.int32(R1)
            D, cbef, tD = scan_hist(r)
            acc = (D ^ 0x80) << 24
            r = r - cbef
            less = cbef

            for level in (1, 2, 3):
                shift = 24 - 8 * level
                mbits = _i32const(0xFFFFFFFF << (shift + 8))
                zero_hist()

                @plsc.parallel_loop(0, NCHUNK, unroll=8)
                def _(i, shift=shift, mbits=mbits, acc=acc):
                    kk = key_v[pl.ds(i * 16, 16)]
                    ing = (kk & mbits) == acc
                    dig = (kk >> shift) & 0xFF
                    plsc.addupdate_scatter(
                        hist_v, [laneoff + dig], ones, mask=ing)
                D, cbef, tD = scan_hist(r)
                acc = acc | (D << shift)
                r = r - cbef
                less = less + cbef

            key_a = acc
            cnt_le = less + tD

            big = jnp.full((16,), 0x7FFFFFFF, jnp.int32)

            @plsc.parallel_loop(0, NCHUNK, unroll=8, carry=big)
            def nm_body(i, acc_v, key_a=key_a):
                kk = key_v[pl.ds(i * 16, 16)]
                return jnp.minimum(acc_v, jnp.where(kk > key_a, kk, big))

            accv = nm_body
            key_b = jnp.min(accv)
            key_b = jnp.where(cnt_le >= jnp.int32(R2 + 1), key_a, key_b)

            va = lax.bitcast_convert_type(
                key_a ^ ((key_a >> 31) & _SIGNMASK), jnp.float32)
            vb = lax.bitcast_convert_type(
                key_b ^ ((key_b >> 31) & _SIGNMASK), jnp.float32)
            m = va + (vb - va) * jnp.float32(FRAC)

            @plsc.parallel_loop(0, NCHUNK, unroll=8)
            def ew_body(i, m=m):
                off = i * 16
                v = row_v[pl.ds(off, 16)]
                row_v[pl.ds(off, 16)] = (
                    jnp.maximum(v - m, jnp.float32(0.0)) + jnp.float32(1.0))

            pltpu.sync_copy(row_v, out_hbm.at[row])

    return sc_kernel(x)


@jax.jit
def kernel(x):
    return _sc_quantile_mask(x)


# speedup vs baseline: 1.7950x; 1.1133x over previous
import functools

import jax
import jax.numpy as jnp
import numpy as np
from jax import lax
from jax.experimental import pallas as pl
from jax.experimental.pallas import tpu as pltpu
from jax.experimental.pallas import tpu_sc as plsc

ROWS = 128
COLS = 32768
NCHUNK = COLS // 16
R1 = 26213
R2 = 26214
FRAC = 0.599609375

NTILES = 32
ROWS_PER_TILE = ROWS // NTILES

_SIGNMASK = 0x7FFFFFFF


def _i32const(v):
    return jnp.int32(np.uint32(v & 0xFFFFFFFF).astype(np.int32))


def _sc_quantile_mask(x):
    mesh = plsc.VectorSubcoreMesh(core_axis_name="c", subcore_axis_name="s")

    @functools.partial(
        pl.kernel,
        mesh=mesh,
        compiler_params=pltpu.CompilerParams(needs_layout_passes=False),
        out_type=jax.ShapeDtypeStruct((ROWS, COLS), jnp.float32),
        scratch_types=[
            pltpu.VMEM((COLS,), jnp.float32),
            pltpu.VMEM((COLS,), jnp.float32),
            pltpu.VMEM((COLS,), jnp.int32),
            pltpu.VMEM((16 * 273,), jnp.int32),
            pltpu.SemaphoreType.DMA,
            pltpu.SemaphoreType.DMA,
        ],
    )
    def sc_kernel(x_hbm, out_hbm, row_a, row_b, key_v, hist_v, sem_in,
                  sem_out):
        wid = lax.axis_index("c") * 16 + lax.axis_index("s")
        lanes = lax.iota(jnp.int32, 16)
        laneoff = lanes * 273
        ones = jnp.full((16,), 1, jnp.int32)
        zeros_i = jnp.zeros((16,), jnp.int32)

        def zero_hist():
            @plsc.parallel_loop(0, 273, unroll=8)
            def _(i):
                hist_v[pl.ds(i * 16, 16)] = zeros_i

        def scan_hist(r):
            z = jnp.int32(0)

            @plsc.parallel_loop(0, 16, unroll=2, carry=(z, z, z, z))
            def scan_body(cb, carry):
                run, D, cbef, tD = carry
                t = hist_v[pl.ds(cb * 16, 16)]
                for l in range(1, 16):
                    t = t + hist_v[pl.ds(l * 273 + cb * 16, 16)]
                c = plsc.cumsum(t) + run
                le = c <= r
                D = D + jnp.sum(jnp.where(le, ones, zeros_i))
                cbef = cbef + jnp.sum(jnp.where(le, t, zeros_i))
                cross = jnp.logical_and(c > r, (c - t) <= r)
                tD = tD + jnp.sum(jnp.where(cross, t, zeros_i))
                run = run + jnp.sum(t)
                return run, D, cbef, tD

            _, D, cbef, tD = scan_body
            return D, cbef, tD

        bufs = [row_a, row_b]
        base = wid * ROWS_PER_TILE
        in_copies = [None] * (ROWS_PER_TILE + 1)
        out_copies = [None] * ROWS_PER_TILE
        in_copies[0] = pltpu.async_copy(x_hbm.at[base], row_a, sem_in)

        for j in range(ROWS_PER_TILE):
            row = base + j
            row_v = bufs[j % 2]
            in_copies[j].wait()

            zero_hist()

            @plsc.parallel_loop(0, NCHUNK, unroll=8)
            def _(i):
                off = i * 16
                v = row_v[pl.ds(off, 16)]
                b = lax.bitcast_convert_type(v, jnp.int32)
                kk = b ^ ((b >> 31) & _SIGNMASK)
                key_v[pl.ds(off, 16)] = kk
                dig = ((kk >> 24) & 0xFF) ^ 0x80
                plsc.addupdate_scatter(hist_v, [laneoff + dig], ones)

            r = jnp.int32(R1)
            D, cbef, tD = scan_hist(r)
            acc = (D ^ 0x80) << 24
            r = r - cbef
            less = cbef

            for level in (1, 2, 3):
                shift = 24 - 8 * level
                mbits = _i32const(0xFFFFFFFF << (shift + 8))
                zero_hist()

                @plsc.parallel_loop(0, NCHUNK, unroll=8)
                def _(i, shift=shift, mbits=mbits, acc=acc):
                    kk = key_v[pl.ds(i * 16, 16)]
                    ing = (kk & mbits) == acc
                    dig = (kk >> shift) & 0xFF
                    plsc.addupdate_scatter(
                        hist_v, [laneoff + dig], ones, mask=ing)
                D, cbef, tD = scan_hist(r)
                acc = acc | (D << shift)
                r = r - cbef
                less = less + cbef

            key_a = acc
            cnt_le = less + tD

            if j + 1 < ROWS_PER_TILE:
                if out_copies[j - 1] is not None:
                    out_copies[j - 1].wait()
                in_copies[j + 1] = pltpu.async_copy(
                    x_hbm.at[row + 1], bufs[(j + 1) % 2], sem_in)

            big = jnp.full((16,), 0x7FFFFFFF, jnp.int32)

            @plsc.parallel_loop(0, NCHUNK, unroll=8, carry=big)
            def nm_body(i, acc_v, key_a=key_a):
                kk = key_v[pl.ds(i * 16, 16)]
                return jnp.minimum(acc_v, jnp.where(kk > key_a, kk, big))

            accv = nm_body
            key_b = jnp.min(accv)
            key_b = jnp.where(cnt_le >= jnp.int32(R2 + 1), key_a, key_b)

            va = lax.bitcast_convert_type(
                key_a ^ ((key_a >> 31) & _SIGNMASK), jnp.float32)
            vb = lax.bitcast_convert_type(
                key_b ^ ((key_b >> 31) & _SIGNMASK), jnp.float32)
            m = va + (vb - va) * jnp.float32(FRAC)

            @plsc.parallel_loop(0, NCHUNK, unroll=8)
            def ew_body(i, m=m):
                off = i * 16
                v = row_v[pl.ds(off, 16)]
                row_v[pl.ds(off, 16)] = (
                    jnp.maximum(v - m, jnp.float32(0.0)) + jnp.float32(1.0))

            out_copies[j] = pltpu.async_copy(row_v, out_hbm.at[row], sem_out)

        out_copies[ROWS_PER_TILE - 2].wait()
        out_copies[ROWS_PER_TILE - 1].wait()

    return sc_kernel(x)


@jax.jit
def kernel(x):
    return _sc_quantile_mask(x)
